# baseline (speedup 1.0000x reference)
import functools

import jax
import jax.numpy as jnp
from jax import lax
from jax.experimental import pallas as pl
from jax.experimental.pallas import tpu as pltpu
from jax.experimental.pallas import tpu_sc as plsc

N, C, H, W = 16, 80, 64, 64
HW = H * W
REG_MAX = 16
P = 2048
W_QFL, W_DFL, W_IOU = 1.0, 0.25, 2.0
EPS = 1e-7

NC, NS = 2, 16
NW = NC * NS
PT = P // NW

_f32 = jnp.float32
_i32 = jnp.int32


def _sc_gather(cls80, dist68, pred3, bid, idx):
    mesh = plsc.VectorSubcoreMesh(core_axis_name="c", subcore_axis_name="s")
    out_type = [
        jax.ShapeDtypeStruct((P, C), _f32),
        jax.ShapeDtypeStruct((P, C), _f32),
        jax.ShapeDtypeStruct((P, 68), _f32),
        jax.ShapeDtypeStruct((P, 4, W), _f32),
    ]

    @functools.partial(
        pl.kernel, mesh=mesh, out_type=out_type,
        compiler_params=pltpu.CompilerParams(use_tc_tiling_on_sc=True),
        scratch_types=[
            pltpu.VMEM((144,), _i32),
            pltpu.VMEM((144,), _i32),
            pltpu.VMEM((PT, C), _f32),
            pltpu.VMEM((PT, C), _f32),
            pltpu.VMEM((PT, 68), _f32),
            pltpu.VMEM((PT, 4, W), _f32),
            pltpu.SemaphoreType.DMA,
            pltpu.SemaphoreType.DMA,
            pltpu.SemaphoreType.DMA,
            pltpu.SemaphoreType.DMA,
        ])
    def k(cls_h, dist_h, pred_h, bid_h, idx_h,
          wout_h, lout_h, dout_h, pout_h,
          b_v, x_v, wbuf, lbuf, dbuf, pbuf, sw, sl, sd, sp):
        wid = lax.axis_index("s") * NC + lax.axis_index("c")
        base = wid * PT
        half = (wid >> 1) * 128
        loc = (wid & 1) * PT
        pltpu.sync_copy(bid_h.at[pl.ds(half, 128)], b_v.at[pl.ds(0, 128)])
        pltpu.sync_copy(idx_h.at[pl.ds(half, 128)], x_v.at[pl.ds(0, 128)])

        @pl.loop(0, PT)
        def _(i):
            bb = b_v[pl.ds(loc + i, 16)][0]
            xx = x_v[pl.ds(loc + i, 16)][0]
            xd = (xx * 52429) >> 22
            r0 = bb * HW
            pltpu.async_copy(cls_h.at[pl.ds(r0 + xd, 1)],
                             wbuf.at[pl.ds(i, 1)], sw)
            pltpu.async_copy(cls_h.at[pl.ds(r0 + xx, 1)],
                             lbuf.at[pl.ds(i, 1)], sl)
            pltpu.async_copy(dist_h.at[pl.ds(r0 + xx, 1)],
                             dbuf.at[pl.ds(i, 1)], sd)
            pltpu.async_copy(pred_h.at[pl.ds(bb * H + (xx >> 6), 1)],
                             pbuf.at[pl.ds(i, 1)], sp)

        pltpu.make_async_copy(wout_h.at[pl.ds(base, PT)], wbuf, sw).wait()
        pltpu.make_async_copy(lout_h.at[pl.ds(base, PT)], lbuf, sl).wait()
        pltpu.make_async_copy(dout_h.at[pl.ds(base, PT)], dbuf, sd).wait()
        pltpu.make_async_copy(pout_h.at[pl.ds(base, PT)], pbuf, sp).wait()

        pltpu.sync_copy(wbuf, wout_h.at[pl.ds(base, PT)])
        pltpu.sync_copy(lbuf, lout_h.at[pl.ds(base, PT)])
        pltpu.sync_copy(dbuf, dout_h.at[pl.ds(base, PT)])
        pltpu.sync_copy(pbuf, pout_h.at[pl.ds(base, PT)])

    return k(cls80, dist68, pred3, bid, idx)


def _dense_body(x_ref, o_ref):
    i = pl.program_id(0)
    l = x_ref[...]
    e = jnp.exp(-jnp.abs(l))
    sp = jnp.maximum(l, 0.0) + jnp.log(1.0 + e)
    sig = jnp.where(l >= 0, 1.0 / (1.0 + e), e / (1.0 + e))
    s = jnp.sum(sp * sig * sig)

    @pl.when(i == 0)
    def _():
        o_ref[...] = jnp.zeros_like(o_ref)

    o_ref[...] = o_ref[...] + s.reshape(1, 1)


def _dense_sum(cls80):
    rows = cls80.shape[0]
    blk = 16384
    return pl.pallas_call(
        _dense_body,
        grid=(rows // blk,),
        in_specs=[pl.BlockSpec((blk, C), lambda i: (i, 0))],
        out_specs=pl.BlockSpec((1, 1), lambda i: (0, 0)),
        out_shape=jax.ShapeDtypeStruct((1, 1), _f32),
    )(cls80)


def _xyxy(cx, cy, w, h):
    return cx - 0.5 * w, cy - 0.5 * h, cx + 0.5 * w, cy + 0.5 * h


def _b1_body(wrow, lrow, drow, parr, idxr, labr, tgt, q_o, l_o, sums_o):
    idv = idxr[...]
    io80 = lax.broadcasted_iota(_i32, (P, C), 1)
    w = jnp.sum(jnp.where(io80 == idv % C, wrow[...], 0.0),
                axis=1, keepdims=True)
    lg = jnp.sum(jnp.where(io80 == labr[...], lrow[...], 0.0),
                 axis=1, keepdims=True)

    io64 = lax.broadcasted_iota(_i32, (P, W), 1)
    wlane = idv % W
    pc = [jnp.sum(jnp.where(io64 == wlane, parr[:, c, :], 0.0),
                  axis=1, keepdims=True) for c in range(4)]
    io4 = lax.broadcasted_iota(_i32, (P, 4), 1)
    tc_ = [jnp.sum(jnp.where(io4 == c, tgt[...], 0.0), axis=1, keepdims=True)
           for c in range(4)]

    ax0, ay0, ax1, ay1 = _xyxy(pc[0], pc[1], pc[2], pc[3])
    bx0, by0, bx1, by1 = _xyxy(tc_[0], tc_[1], tc_[2], tc_[3])
    iw = jnp.maximum(jnp.minimum(ax1, bx1) - jnp.maximum(ax0, bx0), 0.0)
    ih = jnp.maximum(jnp.minimum(ay1, by1) - jnp.maximum(ay0, by0), 0.0)
    inter = iw * ih
    area_a = jnp.maximum(ax1 - ax0, 0.0) * jnp.maximum(ay1 - ay0, 0.0)
    area_b = jnp.maximum(bx1 - bx0, 0.0) * jnp.maximum(by1 - by0, 0.0)
    union = area_a + area_b - inter
    iou = inter / (union + EPS)
    cw = jnp.maximum(jnp.maximum(ax1, bx1) - jnp.minimum(ax0, bx0), 0.0)
    ch = jnp.maximum(jnp.maximum(ay1, by1) - jnp.minimum(ay0, by0), 0.0)
    area_c = cw * ch
    giou_l = 1.0 - (iou - (area_c - union) / (area_c + EPS))

    x68 = drow[...]
    io68 = lax.broadcasted_iota(_i32, (P, 68), 1)
    dfl_acc = jnp.zeros((P, 1), _f32)
    for s in range(4):
        lo = 17 * s
        mask = (io68 >= lo) & (io68 < lo + 17)
        label = jnp.clip(tc_[s] * REG_MAX, 0.0, REG_MAX - 1e-4)
        dl = jnp.floor(label)
        dli = dl.astype(_i32)
        wl = dl + 1.0 - label
        wr_ = label - dl
        m = jnp.max(jnp.where(mask, x68, -1e30), axis=1, keepdims=True)
        lse = jnp.log(jnp.sum(jnp.where(mask, jnp.exp(x68 - m), 0.0),
                              axis=1, keepdims=True)) + m
        xdl = jnp.sum(jnp.where(io68 == lo + dli, x68, 0.0),
                      axis=1, keepdims=True)
        xdr = jnp.sum(jnp.where(io68 == lo + dli + 1, x68, 0.0),
                      axis=1, keepdims=True)
        dfl_acc = dfl_acc + ((lse - xdl) * wl + (lse - xdr) * wr_)

    q_o[...] = iou
    l_o[...] = lg
    sums_o[...] = jnp.concatenate(
        [jnp.sum(w).reshape(1, 1),
         jnp.sum(giou_l * w).reshape(1, 1),
         jnp.sum(dfl_acc * w).reshape(1, 1),
         jnp.zeros((1, 1), _f32)], axis=1)


def _b1_call(wrow, lrow, drow, parr, idxr, labr, tgt):
    return pl.pallas_call(
        _b1_body,
        out_shape=[jax.ShapeDtypeStruct((P, 1), _f32),
                   jax.ShapeDtypeStruct((P, 1), _f32),
                   jax.ShapeDtypeStruct((1, 4), _f32)],
    )(wrow, lrow, drow, parr, idxr, labr, tgt)


_BLK = 512


def _b2_body(bidr, idxr, labr, bidc, idxc, labc, qc, lr, dense, nbp, sums, o_ref):
    keyc = bidc[...] * HW + idxc[...]
    tkeyc = keyc * C + labc[...]
    iop = lax.broadcasted_iota(_i32, (_BLK, P), 1)
    corr = jnp.zeros((1, 1), _f32)
    for rb in range(P // _BLK):
        rs = slice(rb * _BLK, (rb + 1) * _BLK)
        keyr = bidr[rs, :] * HW + idxr[rs, :]
        tkeyr = keyr * C + labr[rs, :]
        eq = keyr == keyc
        win = jnp.max(jnp.where(eq, iop, -1), axis=1, keepdims=True)
        qwin = jnp.sum(jnp.where(iop == win, qc[...], 0.0),
                       axis=1, keepdims=True)
        twin = jnp.max(jnp.where(tkeyr == tkeyc, iop, -1),
                       axis=1, keepdims=True)
        rowid = rb * _BLK + lax.broadcasted_iota(_i32, (_BLK, 1), 0)
        trep = twin == rowid
        l = lr[rs, :]
        e = jnp.exp(-jnp.abs(l))
        sp = jnp.maximum(l, 0.0) + jnp.log(1.0 + e)
        sig = jnp.where(l >= 0, 1.0 / (1.0 + e), e / (1.0 + e))
        d = sig - qwin
        term = (sp - l * qwin) * d * d
        term0 = sp * sig * sig
        corr = corr + jnp.sum(jnp.where(trep, term - term0, 0.0)).reshape(1, 1)

    nb = jnp.maximum(nbp[0, 0], 1.0)
    wt_sum = sums[0, 0]
    loss_qfl = (dense[0, 0] + corr[0, 0]) / nb
    loss_iou = sums[0, 1] / wt_sum
    loss_dfl = (sums[0, 2] / 4.0) / wt_sum
    qfl = W_QFL * loss_qfl
    dfl = W_DFL * loss_dfl
    iou = W_IOU * loss_iou
    total = qfl + dfl + iou
    o_ref[...] = jnp.concatenate(
        [total.reshape(1, 1), qfl.reshape(1, 1),
         dfl.reshape(1, 1), iou.reshape(1, 1)], axis=1)


def _b2_call(bidr, idxr, labr, bidc, idxc, labc, qc, lr, dense, nbp, sums):
    return pl.pallas_call(
        _b2_body,
        out_shape=jax.ShapeDtypeStruct((1, 4), _f32),
    )(bidr, idxr, labr, bidc, idxc, labc, qc, lr, dense, nbp, sums)


def kernel(cls_score, predicted_bbox, bbox_distribution, num_boxes_pos,
           target_feat_map_indices_batch_id_vector, target_feat_map_indices,
           target_class_label_vector, target_bounding_box_label_matrix):
    bid = target_feat_map_indices_batch_id_vector
    idx = target_feat_map_indices
    lab = target_class_label_vector
    tgt = target_bounding_box_label_matrix

    cls80 = cls_score.transpose(0, 2, 3, 1).reshape(N * HW, C)
    dist68 = bbox_distribution.reshape(N * HW, 4 * (REG_MAX + 1))
    pred3 = predicted_bbox.transpose(0, 1, 3, 2).reshape(N * H, 4, W)

    wrow, lrow, drow, parr = _sc_gather(cls80, dist68, pred3, bid, idx)

    dense = _dense_sum(cls80)

    idxr = idx.reshape(P, 1)
    labr = lab.reshape(P, 1)
    q, lg, sums = _b1_call(wrow, lrow, drow, parr, idxr, labr, tgt)

    out = _b2_call(bid.reshape(P, 1), idxr, labr,
                   bid.reshape(1, P), idx.reshape(1, P), lab.reshape(1, P),
                   q.reshape(1, P), lg, dense,
                   num_boxes_pos.reshape(1, 1), sums)
    return (out[0, 0], out[0, 1], out[0, 2], out[0, 3])


# speedup vs baseline: 1.0056x; 1.0056x over previous
import functools

import jax
import jax.numpy as jnp
from jax import lax
from jax.experimental import pallas as pl
from jax.experimental.pallas import tpu as pltpu
from jax.experimental.pallas import tpu_sc as plsc

N, C, H, W = 16, 80, 64, 64
HW = H * W
REG_MAX = 16
P = 2048
W_QFL, W_DFL, W_IOU = 1.0, 0.25, 2.0
EPS = 1e-7

NC, NS = 2, 16
NW = NC * NS
PT = P // NW

_f32 = jnp.float32
_i32 = jnp.int32


def _sc_gather(cls80, dist68, pred3, bid, idx):
    mesh = plsc.VectorSubcoreMesh(core_axis_name="c", subcore_axis_name="s")
    out_type = [
        jax.ShapeDtypeStruct((P, C), _f32),
        jax.ShapeDtypeStruct((P, C), _f32),
        jax.ShapeDtypeStruct((P, 68), _f32),
        jax.ShapeDtypeStruct((P, 4, W), _f32),
    ]

    @functools.partial(
        pl.kernel, mesh=mesh, out_type=out_type,
        compiler_params=pltpu.CompilerParams(use_tc_tiling_on_sc=True),
        scratch_types=[
            pltpu.VMEM((144,), _i32),
            pltpu.VMEM((144,), _i32),
            pltpu.VMEM((PT, C), _f32),
            pltpu.VMEM((PT, C), _f32),
            pltpu.VMEM((PT, 68), _f32),
            pltpu.VMEM((PT, 4, W), _f32),
            pltpu.SemaphoreType.DMA,
            pltpu.SemaphoreType.DMA,
            pltpu.SemaphoreType.DMA,
            pltpu.SemaphoreType.DMA,
        ])
    def k(cls_h, dist_h, pred_h, bid_h, idx_h,
          wout_h, lout_h, dout_h, pout_h,
          b_v, x_v, wbuf, lbuf, dbuf, pbuf, sw, sl, sd, sp):
        wid = lax.axis_index("s") * NC + lax.axis_index("c")
        base = wid * PT
        half = (wid >> 1) * 128
        loc = (wid & 1) * PT
        pltpu.sync_copy(bid_h.at[pl.ds(half, 128)], b_v.at[pl.ds(0, 128)])
        pltpu.sync_copy(idx_h.at[pl.ds(half, 128)], x_v.at[pl.ds(0, 128)])

        @pl.loop(0, PT)
        def _(i):
            bb = b_v[pl.ds(loc + i, 16)][0]
            xx = x_v[pl.ds(loc + i, 16)][0]
            xd = (xx * 52429) >> 22
            r0 = bb * HW
            pltpu.async_copy(cls_h.at[pl.ds(r0 + xd, 1)],
                             wbuf.at[pl.ds(i, 1)], sw)
            pltpu.async_copy(cls_h.at[pl.ds(r0 + xx, 1)],
                             lbuf.at[pl.ds(i, 1)], sl)
            pltpu.async_copy(dist_h.at[pl.ds(r0 + xx, 1)],
                             dbuf.at[pl.ds(i, 1)], sd)
            pltpu.async_copy(pred_h.at[pl.ds(bb * H + (xx >> 6), 1)],
                             pbuf.at[pl.ds(i, 1)], sp)

        pltpu.make_async_copy(wout_h.at[pl.ds(base, PT)], wbuf, sw).wait()
        pltpu.make_async_copy(lout_h.at[pl.ds(base, PT)], lbuf, sl).wait()
        pltpu.make_async_copy(dout_h.at[pl.ds(base, PT)], dbuf, sd).wait()
        pltpu.make_async_copy(pout_h.at[pl.ds(base, PT)], pbuf, sp).wait()

        pltpu.sync_copy(wbuf, wout_h.at[pl.ds(base, PT)])
        pltpu.sync_copy(lbuf, lout_h.at[pl.ds(base, PT)])
        pltpu.sync_copy(dbuf, dout_h.at[pl.ds(base, PT)])
        pltpu.sync_copy(pbuf, pout_h.at[pl.ds(base, PT)])

    return k(cls80, dist68, pred3, bid, idx)


def _dense_body(x_ref, o_ref):
    i = pl.program_id(0)
    l = x_ref[...]
    e = jnp.exp(-jnp.abs(l))
    sp = jnp.maximum(l, 0.0) + jnp.log(1.0 + e)
    sig = jnp.where(l >= 0, 1.0 / (1.0 + e), e / (1.0 + e))
    s = jnp.sum(sp * sig * sig)

    @pl.when(i == 0)
    def _():
        o_ref[...] = jnp.zeros_like(o_ref)

    o_ref[...] = o_ref[...] + s.reshape(1, 1)


def _dense_sum(cls80):
    rows = cls80.shape[0]
    blk = 8192
    return pl.pallas_call(
        _dense_body,
        grid=(rows // blk,),
        in_specs=[pl.BlockSpec((blk, C), lambda i: (i, 0))],
        out_specs=pl.BlockSpec((1, 1), lambda i: (0, 0)),
        out_shape=jax.ShapeDtypeStruct((1, 1), _f32),
    )(cls80)


def _xyxy(cx, cy, w, h):
    return cx - 0.5 * w, cy - 0.5 * h, cx + 0.5 * w, cy + 0.5 * h


def _b1_body(wrow, lrow, drow, parr, idxr, labr, tgt, q_o, l_o, sums_o):
    idv = idxr[...]
    io80 = lax.broadcasted_iota(_i32, (P, C), 1)
    w = jnp.sum(jnp.where(io80 == idv % C, wrow[...], 0.0),
                axis=1, keepdims=True)
    lg = jnp.sum(jnp.where(io80 == labr[...], lrow[...], 0.0),
                 axis=1, keepdims=True)

    io64 = lax.broadcasted_iota(_i32, (P, W), 1)
    wlane = idv % W
    pc = [jnp.sum(jnp.where(io64 == wlane, parr[:, c, :], 0.0),
                  axis=1, keepdims=True) for c in range(4)]
    io4 = lax.broadcasted_iota(_i32, (P, 4), 1)
    tc_ = [jnp.sum(jnp.where(io4 == c, tgt[...], 0.0), axis=1, keepdims=True)
           for c in range(4)]

    ax0, ay0, ax1, ay1 = _xyxy(pc[0], pc[1], pc[2], pc[3])
    bx0, by0, bx1, by1 = _xyxy(tc_[0], tc_[1], tc_[2], tc_[3])
    iw = jnp.maximum(jnp.minimum(ax1, bx1) - jnp.maximum(ax0, bx0), 0.0)
    ih = jnp.maximum(jnp.minimum(ay1, by1) - jnp.maximum(ay0, by0), 0.0)
    inter = iw * ih
    area_a = jnp.maximum(ax1 - ax0, 0.0) * jnp.maximum(ay1 - ay0, 0.0)
    area_b = jnp.maximum(bx1 - bx0, 0.0) * jnp.maximum(by1 - by0, 0.0)
    union = area_a + area_b - inter
    iou = inter / (union + EPS)
    cw = jnp.maximum(jnp.maximum(ax1, bx1) - jnp.minimum(ax0, bx0), 0.0)
    ch = jnp.maximum(jnp.maximum(ay1, by1) - jnp.minimum(ay0, by0), 0.0)
    area_c = cw * ch
    giou_l = 1.0 - (iou - (area_c - union) / (area_c + EPS))

    x68 = drow[...]
    io68 = lax.broadcasted_iota(_i32, (P, 68), 1)
    dfl_acc = jnp.zeros((P, 1), _f32)
    for s in range(4):
        lo = 17 * s
        mask = (io68 >= lo) & (io68 < lo + 17)
        label = jnp.clip(tc_[s] * REG_MAX, 0.0, REG_MAX - 1e-4)
        dl = jnp.floor(label)
        dli = dl.astype(_i32)
        wl = dl + 1.0 - label
        wr_ = label - dl
        m = jnp.max(jnp.where(mask, x68, -1e30), axis=1, keepdims=True)
        lse = jnp.log(jnp.sum(jnp.where(mask, jnp.exp(x68 - m), 0.0),
                              axis=1, keepdims=True)) + m
        xdl = jnp.sum(jnp.where(io68 == lo + dli, x68, 0.0),
                      axis=1, keepdims=True)
        xdr = jnp.sum(jnp.where(io68 == lo + dli + 1, x68, 0.0),
                      axis=1, keepdims=True)
        dfl_acc = dfl_acc + ((lse - xdl) * wl + (lse - xdr) * wr_)

    q_o[...] = iou
    l_o[...] = lg
    sums_o[...] = jnp.concatenate(
        [jnp.sum(w).reshape(1, 1),
         jnp.sum(giou_l * w).reshape(1, 1),
         jnp.sum(dfl_acc * w).reshape(1, 1),
         jnp.zeros((1, 1), _f32)], axis=1)


def _b1_call(wrow, lrow, drow, parr, idxr, labr, tgt):
    return pl.pallas_call(
        _b1_body,
        out_shape=[jax.ShapeDtypeStruct((P, 1), _f32),
                   jax.ShapeDtypeStruct((P, 1), _f32),
                   jax.ShapeDtypeStruct((1, 4), _f32)],
    )(wrow, lrow, drow, parr, idxr, labr, tgt)


_BLK = 256


def _b2_body(bidr, idxr, labr, bidc, idxc, labc, qc, lr, dense, nbp, sums, o_ref):
    keyc = bidc[...] * HW + idxc[...]
    tkeyc = keyc * C + labc[...]
    iop = lax.broadcasted_iota(_i32, (_BLK, P), 1)
    corr = jnp.zeros((1, 1), _f32)
    for rb in range(P // _BLK):
        rs = slice(rb * _BLK, (rb + 1) * _BLK)
        keyr = bidr[rs, :] * HW + idxr[rs, :]
        tkeyr = keyr * C + labr[rs, :]
        eq = keyr == keyc
        win = jnp.max(jnp.where(eq, iop, -1), axis=1, keepdims=True)
        qwin = jnp.sum(jnp.where(iop == win, qc[...], 0.0),
                       axis=1, keepdims=True)
        twin = jnp.max(jnp.where(tkeyr == tkeyc, iop, -1),
                       axis=1, keepdims=True)
        rowid = rb * _BLK + lax.broadcasted_iota(_i32, (_BLK, 1), 0)
        trep = twin == rowid
        l = lr[rs, :]
        e = jnp.exp(-jnp.abs(l))
        sp = jnp.maximum(l, 0.0) + jnp.log(1.0 + e)
        sig = jnp.where(l >= 0, 1.0 / (1.0 + e), e / (1.0 + e))
        d = sig - qwin
        term = (sp - l * qwin) * d * d
        term0 = sp * sig * sig
        corr = corr + jnp.sum(jnp.where(trep, term - term0, 0.0)).reshape(1, 1)

    nb = jnp.maximum(nbp[0, 0], 1.0)
    wt_sum = sums[0, 0]
    loss_qfl = (dense[0, 0] + corr[0, 0]) / nb
    loss_iou = sums[0, 1] / wt_sum
    loss_dfl = (sums[0, 2] / 4.0) / wt_sum
    qfl = W_QFL * loss_qfl
    dfl = W_DFL * loss_dfl
    iou = W_IOU * loss_iou
    total = qfl + dfl + iou
    o_ref[...] = jnp.concatenate(
        [total.reshape(1, 1), qfl.reshape(1, 1),
         dfl.reshape(1, 1), iou.reshape(1, 1)], axis=1)


def _b2_call(bidr, idxr, labr, bidc, idxc, labc, qc, lr, dense, nbp, sums):
    return pl.pallas_call(
        _b2_body,
        out_shape=jax.ShapeDtypeStruct((1, 4), _f32),
    )(bidr, idxr, labr, bidc, idxc, labc, qc, lr, dense, nbp, sums)


def kernel(cls_score, predicted_bbox, bbox_distribution, num_boxes_pos,
           target_feat_map_indices_batch_id_vector, target_feat_map_indices,
           target_class_label_vector, target_bounding_box_label_matrix):
    bid = target_feat_map_indices_batch_id_vector
    idx = target_feat_map_indices
    lab = target_class_label_vector
    tgt = target_bounding_box_label_matrix

    cls80 = cls_score.transpose(0, 2, 3, 1).reshape(N * HW, C)
    dist68 = bbox_distribution.reshape(N * HW, 4 * (REG_MAX + 1))
    pred3 = predicted_bbox.transpose(0, 1, 3, 2).reshape(N * H, 4, W)

    wrow, lrow, drow, parr = _sc_gather(cls80, dist68, pred3, bid, idx)

    dense = _dense_sum(cls80)

    idxr = idx.reshape(P, 1)
    labr = lab.reshape(P, 1)
    q, lg, sums = _b1_call(wrow, lrow, drow, parr, idxr, labr, tgt)

    out = _b2_call(bid.reshape(P, 1), idxr, labr,
                   bid.reshape(1, P), idx.reshape(1, P), lab.reshape(1, P),
                   q.reshape(1, P), lg, dense,
                   num_boxes_pos.reshape(1, 1), sums)
    return (out[0, 0], out[0, 1], out[0, 2], out[0, 3])
